# baseline (speedup 1.0000x reference)
import jax
import jax.numpy as jnp
from jax import lax
from jax.experimental import pallas as pl
from jax.experimental.pallas import tpu as pltpu
from jax.experimental.pallas import tpu_sc as plsc

UNIQUE_TOKENS = 1000
CONTEXT = 2
EMB = 256
HID = 1024
B = 1024
T = 50
NC = T - 2 * CONTEXT
NSLOT = 8
TCUTS = (0, 12, 24, T)
CCUTS = (0, 8, 20, NC)

_SC_INFO = plsc.get_sparse_core_info()
NCORE = _SC_INFO.num_cores
NSUB = _SC_INFO.num_subcores
NW = NCORE * NSUB
CHUNK = 64


def _make_gather(nslabs):
    nrows = nslabs * B
    rpw = nrows // NW
    nch = rpw // CHUNK

    def body(idx_hbm, table_hbm, out_hbm, idx_v, buf0, buf1, sem0, sem1):
        wid = lax.axis_index("s") * NCORE + lax.axis_index("c")
        pltpu.sync_copy(idx_hbm.at[wid], idx_v)
        bufs = (buf0, buf1)
        sems = (sem0, sem1)
        handles = [None] * nch
        handles[0] = pltpu.async_copy(table_hbm.at[idx_v.at[0]], buf0, sem0)
        for c in range(nch):
            if c + 1 < nch:
                handles[c + 1] = pltpu.async_copy(
                    table_hbm.at[idx_v.at[c + 1]], bufs[(c + 1) % 2], sems[(c + 1) % 2]
                )
            handles[c].wait()
            row0 = pl.multiple_of(wid * rpw + c * CHUNK, 8)
            pltpu.sync_copy(bufs[c % 2], out_hbm.at[pl.ds(row0, CHUNK)])

    return pl.kernel(
        body,
        mesh=plsc.VectorSubcoreMesh(core_axis_name="c", subcore_axis_name="s"),
        out_type=jax.ShapeDtypeStruct((nrows, EMB), jnp.float32),
        scratch_types=[
            pltpu.VMEM((nch, CHUNK), jnp.int32),
            pltpu.VMEM((CHUNK, EMB), jnp.float32),
            pltpu.VMEM((CHUNK, EMB), jnp.float32),
            pltpu.SemaphoreType.DMA,
            pltpu.SemaphoreType.DMA,
        ],
    )


_gathers = [_make_gather(TCUTS[k + 1] - TCUTS[k]) for k in range(3)]


def _slab_copy(e_hbm, ring, sems, t, base=0):
    return pltpu.make_async_copy(
        e_hbm.at[t - base], ring.at[t % NSLOT], sems.at[t % NSLOT]
    )


def _two_center_mlp(w1_ref, b1_ref, w2_ref, b2_ref, out_ref, ring, t0):
    shared = ring[(t0 + 1) % NSLOT] + ring[(t0 + 4) % NSLOT]
    ctx0 = (shared + ring[t0 % NSLOT] + ring[(t0 + 3) % NSLOT]).astype(jnp.bfloat16)
    ctx1 = (shared + ring[(t0 + 2) % NSLOT] + ring[(t0 + 5) % NSLOT]).astype(jnp.bfloat16)

    ctx = jnp.concatenate([ctx0, ctx1], axis=0)
    h = jax.lax.dot_general(
        w1_ref[...], ctx, (((1,), (1,)), ((), ())),
        preferred_element_type=jnp.float32,
    )
    r = jnp.maximum(h + b1_ref[...], 0.0).astype(jnp.bfloat16)
    out = jax.lax.dot_general(
        w2_ref[...], r, (((1,), (0,)), ((), ())),
        preferred_element_type=jnp.float32,
    )
    out_ref[0] = out[:, :B] + b2_ref[...]
    out_ref[1] = out[:, B:] + b2_ref[...]


def _make_mlp_body(c_lo, c_hi, slab_ranges, n_dummy):
    t_need_max = c_hi + 3

    def body(*args):
        e_refs = args[n_dummy:n_dummy + len(slab_ranges)]
        w1_ref, b1_ref, w2_ref, b2_ref, out_ref, ring, sems = args[
            n_dummy + len(slab_ranges):
        ]
        j = pl.program_id(0)
        t0 = c_lo + 2 * j

        def src_for(t):
            for (lo, hi), ref in zip(slab_ranges, e_refs):
                if lo <= t < hi:
                    return ref, lo
            raise AssertionError(t)

        @pl.when(j == 0)
        def _prime():
            for t in range(c_lo, c_lo + 6):
                ref, base = src_for(t)
                _slab_copy(ref, ring, sems, t, base).start()
            for t in range(c_lo, c_lo + 6):
                ref, base = src_for(t)
                _slab_copy(ref, ring, sems, t, base).wait()

        last_ref = e_refs[-1]
        last_lo = slab_ranges[-1][0]
        assert c_lo + 6 >= last_lo

        @pl.when(t0 + 7 <= t_need_max)
        def _prefetch():
            _slab_copy(last_ref, ring, sems, t0 + 6, last_lo).start()
            _slab_copy(last_ref, ring, sems, t0 + 7, last_lo).start()

        @pl.when(j > 0)
        def _await_slab():
            _slab_copy(last_ref, ring, sems, t0 + 4, last_lo).wait()
            _slab_copy(last_ref, ring, sems, t0 + 5, last_lo).wait()

        _two_center_mlp(w1_ref, b1_ref, w2_ref, b2_ref, out_ref, ring, t0)

    return body


_body1 = _make_mlp_body(CCUTS[0], CCUTS[1], [(TCUTS[0], TCUTS[1])], 0)
_body2 = _make_mlp_body(
    CCUTS[1], CCUTS[2], [(TCUTS[0], TCUTS[1]), (TCUTS[1], TCUTS[2])], 1
)
_body3 = _make_mlp_body(
    CCUTS[2], CCUTS[3], [(TCUTS[1], TCUTS[2]), (TCUTS[2], TCUTS[3])], 1
)


@jax.jit
def kernel(batchCode, table, W1, b1, W2, b2):
    idx_t = batchCode.astype(jnp.int32).T
    es = []
    for k in range(3):
        lo, hi = TCUTS[k], TCUTS[k + 1]
        idx = idx_t[lo:hi].reshape(NW, -1, CHUNK)
        es.append(_gathers[k](idx, table).reshape(hi - lo, B, EMB))
    e1, e2, e3 = es

    w1t = W1.T.astype(jnp.bfloat16)
    b1c = b1.reshape(HID, 1)
    w2t = W2.T.astype(jnp.bfloat16)
    b2c = b2.reshape(UNIQUE_TOKENS, 1)
    weights = (w1t, b1c, w2t, b2c)
    wspecs = [
        pl.BlockSpec((HID, EMB), lambda i: (0, 0)),
        pl.BlockSpec((HID, 1), lambda i: (0, 0)),
        pl.BlockSpec((UNIQUE_TOKENS, HID), lambda i: (0, 0)),
        pl.BlockSpec((UNIQUE_TOKENS, 1), lambda i: (0, 0)),
    ]
    out_shape = jax.ShapeDtypeStruct((NC, UNIQUE_TOKENS, B), jnp.float32)

    def mlp_pass(body, n_e, c_lo, c_hi, alias, operands):
        scratch = [
            pltpu.VMEM((NSLOT, B, EMB), jnp.float32),
            pltpu.SemaphoreType.DMA((NSLOT,)),
        ]
        n_dummy = 1 if alias else 0
        return pl.pallas_call(
            body,
            grid=((c_hi - c_lo) // 2,),
            in_specs=[pl.BlockSpec(memory_space=pl.ANY)] * (n_dummy + n_e)
            + wspecs,
            out_specs=pl.BlockSpec(
                (2, UNIQUE_TOKENS, B), lambda i, base=c_lo // 2: (i + base, 0, 0)
            ),
            out_shape=out_shape,
            input_output_aliases={0: 0} if alias else {},
            scratch_shapes=scratch,
        )(*operands)

    part1 = mlp_pass(_body1, 1, CCUTS[0], CCUTS[1], False, (e1, *weights))
    part2 = mlp_pass(
        _body2, 2, CCUTS[1], CCUTS[2], True, (part1, e1, e2, *weights)
    )
    out_t = mlp_pass(
        _body3, 2, CCUTS[2], CCUTS[3], True, (part2, e2, e3, *weights)
    )

    return jnp.transpose(out_t, (2, 0, 1))


# speedup vs baseline: 1.0120x; 1.0120x over previous
import functools

import jax
import jax.numpy as jnp
from jax import lax
from jax.experimental import pallas as pl
from jax.experimental.pallas import tpu as pltpu
from jax.experimental.pallas import tpu_sc as plsc

UNIQUE_TOKENS = 1000
CONTEXT = 2
EMB = 256
HID = 1024
B = 1024
T = 50
NC = T - 2 * CONTEXT
NSLOT = 8
TSPLIT = 24
CSPLIT = 20

_SC_INFO = plsc.get_sparse_core_info()
NCORE = _SC_INFO.num_cores
NSUB = _SC_INFO.num_subcores
NW = NCORE * NSUB
CHUNK = 64


def _make_gather(nslabs):
    nrows = nslabs * B
    rpw = nrows // NW
    nch = rpw // CHUNK

    def body(idx_hbm, table_hbm, out_hbm, idx_v, buf0, buf1, sem0, sem1):
        wid = lax.axis_index("s") * NCORE + lax.axis_index("c")
        pltpu.sync_copy(idx_hbm.at[wid], idx_v)
        bufs = (buf0, buf1)
        sems = (sem0, sem1)
        handles = [None] * nch
        handles[0] = pltpu.async_copy(table_hbm.at[idx_v.at[0]], buf0, sem0)
        for c in range(nch):
            if c + 1 < nch:
                handles[c + 1] = pltpu.async_copy(
                    table_hbm.at[idx_v.at[c + 1]], bufs[(c + 1) % 2], sems[(c + 1) % 2]
                )
            handles[c].wait()
            row0 = pl.multiple_of(wid * rpw + c * CHUNK, 8)
            pltpu.sync_copy(bufs[c % 2], out_hbm.at[pl.ds(row0, CHUNK)])

    return pl.kernel(
        body,
        mesh=plsc.VectorSubcoreMesh(core_axis_name="c", subcore_axis_name="s"),
        out_type=jax.ShapeDtypeStruct((nrows, EMB), jnp.float32),
        scratch_types=[
            pltpu.VMEM((nch, CHUNK), jnp.int32),
            pltpu.VMEM((CHUNK, EMB), jnp.float32),
            pltpu.VMEM((CHUNK, EMB), jnp.float32),
            pltpu.SemaphoreType.DMA,
            pltpu.SemaphoreType.DMA,
        ],
    )


_gather1 = _make_gather(TSPLIT)
_gather2 = _make_gather(T - TSPLIT)


def _slab_copy(e_hbm, ring, sems, t, base=0):
    return pltpu.make_async_copy(
        e_hbm.at[t - base], ring.at[t % NSLOT], sems.at[t % NSLOT]
    )


def _two_center_mlp(w1_ref, b1_ref, w2_ref, b2_ref, out_ref, ring, t0):
    shared = ring[(t0 + 1) % NSLOT] + ring[(t0 + 4) % NSLOT]
    ctx0 = (shared + ring[t0 % NSLOT] + ring[(t0 + 3) % NSLOT]).astype(jnp.bfloat16)
    ctx1 = (shared + ring[(t0 + 2) % NSLOT] + ring[(t0 + 5) % NSLOT]).astype(jnp.bfloat16)

    ctx = jnp.concatenate([ctx0, ctx1], axis=0)
    h = jax.lax.dot_general(
        w1_ref[...], ctx, (((1,), (1,)), ((), ())),
        preferred_element_type=jnp.float32,
    )
    r = jnp.maximum(h + b1_ref[...], 0.0).astype(jnp.bfloat16)
    out = jax.lax.dot_general(
        w2_ref[...], r, (((1,), (0,)), ((), ())),
        preferred_element_type=jnp.float32,
    )
    out_ref[0] = out[:, :B] + b2_ref[...]
    out_ref[1] = out[:, B:] + b2_ref[...]


def _mlp_body1(e1_hbm, w1_ref, b1_ref, w2_ref, b2_ref, out_ref, ring, sems):
    j = pl.program_id(0)
    t0 = 2 * j

    @pl.when(j == 0)
    def _prime():
        for t in range(6):
            _slab_copy(e1_hbm, ring, sems, t).start()
        for t in range(6):
            _slab_copy(e1_hbm, ring, sems, t).wait()

    @pl.when(t0 + 7 < TSPLIT)
    def _prefetch():
        _slab_copy(e1_hbm, ring, sems, t0 + 6).start()
        _slab_copy(e1_hbm, ring, sems, t0 + 7).start()

    @pl.when(j > 0)
    def _await_slab():
        _slab_copy(e1_hbm, ring, sems, t0 + 4).wait()
        _slab_copy(e1_hbm, ring, sems, t0 + 5).wait()

    _two_center_mlp(w1_ref, b1_ref, w2_ref, b2_ref, out_ref, ring, t0)


def _mlp_body2(out_hbm, e1_hbm, e2_hbm, w1_ref, b1_ref, w2_ref, b2_ref,
               out_ref, ring, sems):
    del out_hbm
    j = pl.program_id(0)
    t0 = CSPLIT + 2 * j

    @pl.when(j == 0)
    def _prime():
        for t in range(CSPLIT, CSPLIT + 6):
            src, base = (e1_hbm, 0) if t < TSPLIT else (e2_hbm, TSPLIT)
            _slab_copy(src, ring, sems, t, base).start()
        for t in range(CSPLIT, CSPLIT + 6):
            src, base = (e1_hbm, 0) if t < TSPLIT else (e2_hbm, TSPLIT)
            _slab_copy(src, ring, sems, t, base).wait()

    @pl.when(t0 + 7 < T)
    def _prefetch():
        _slab_copy(e2_hbm, ring, sems, t0 + 6, TSPLIT).start()
        _slab_copy(e2_hbm, ring, sems, t0 + 7, TSPLIT).start()

    @pl.when(j > 0)
    def _await_slab():
        _slab_copy(e2_hbm, ring, sems, t0 + 4, TSPLIT).wait()
        _slab_copy(e2_hbm, ring, sems, t0 + 5, TSPLIT).wait()

    _two_center_mlp(w1_ref, b1_ref, w2_ref, b2_ref, out_ref, ring, t0)


@jax.jit
def kernel(batchCode, table, W1, b1, W2, b2):
    idx_t = batchCode.astype(jnp.int32).T
    idx1 = idx_t[:TSPLIT].reshape(NW, -1, CHUNK)
    idx2 = idx_t[TSPLIT:].reshape(NW, -1, CHUNK)
    e1 = _gather1(idx1, table).reshape(TSPLIT, B, EMB)
    e2 = _gather2(idx2, table).reshape(T - TSPLIT, B, EMB)

    w1t = W1.T.astype(jnp.bfloat16)
    b1c = b1.reshape(HID, 1)
    w2t = W2.T.astype(jnp.bfloat16)
    b2c = b2.reshape(UNIQUE_TOKENS, 1)
    wspecs = [
        pl.BlockSpec((HID, EMB), lambda i: (0, 0)),
        pl.BlockSpec((HID, 1), lambda i: (0, 0)),
        pl.BlockSpec((UNIQUE_TOKENS, HID), lambda i: (0, 0)),
        pl.BlockSpec((UNIQUE_TOKENS, 1), lambda i: (0, 0)),
    ]
    scratch = [
        pltpu.VMEM((NSLOT, B, EMB), jnp.float32),
        pltpu.SemaphoreType.DMA((NSLOT,)),
    ]
    out_shape = jax.ShapeDtypeStruct((NC, UNIQUE_TOKENS, B), jnp.float32)

    part1 = pl.pallas_call(
        _mlp_body1,
        grid=(CSPLIT // 2,),
        in_specs=[pl.BlockSpec(memory_space=pl.ANY)] + wspecs,
        out_specs=pl.BlockSpec((2, UNIQUE_TOKENS, B), lambda i: (i, 0, 0)),
        out_shape=out_shape,
        scratch_shapes=scratch,
    )(e1, w1t, b1c, w2t, b2c)

    out_t = pl.pallas_call(
        _mlp_body2,
        grid=((NC - CSPLIT) // 2,),
        in_specs=[pl.BlockSpec(memory_space=pl.ANY)] * 3 + wspecs,
        out_specs=pl.BlockSpec(
            (2, UNIQUE_TOKENS, B), lambda i: (i + CSPLIT // 2, 0, 0)
        ),
        out_shape=out_shape,
        input_output_aliases={0: 0},
        scratch_shapes=scratch,
    )(part1, e1, e2, w1t, b1c, w2t, b2c)

    return jnp.transpose(out_t, (2, 0, 1))
